# baseline (speedup 1.0000x reference)
import functools

import jax
import jax.numpy as jnp
from jax import lax
from jax.experimental import pallas as pl
from jax.experimental.pallas import tpu as pltpu
from jax.experimental.pallas import tpu_sc as plsc

B = 16384
N = 1000000
NC = 2
NS = 16
NW = NC * NS
RP = 31264
NT = NW * RP
BW = B // NW
CH = 128
NCH = BW // CH

_mesh = plsc.VectorSubcoreMesh(core_axis_name="c", subcore_axis_name="s")


@functools.partial(
    pl.kernel,
    out_type=jax.ShapeDtypeStruct((NT,), jnp.int32),
    mesh=_mesh,
    scratch_types=[
        pltpu.VMEM((B,), jnp.int32),
        pltpu.VMEM((RP,), jnp.int32),
    ],
    compiler_params=pltpu.CompilerParams(needs_layout_passes=False),
)
def _winners(idx_hbm, table_hbm, idx_v, table_v):
    wid = lax.axis_index("s") * NC + lax.axis_index("c")
    lo = wid * RP
    pltpu.sync_copy(idx_hbm, idx_v)
    iota = lax.iota(jnp.int32, 16)

    def body(v, carry):
        iv = idx_v[pl.ds(v * 16, 16)]
        jv = v * 16 + iota
        loc = iv - lo
        m = (loc >= 0) & (loc < RP)
        plsc.store_scatter(table_v, [loc], jv, mask=m)
        got = plsc.load_gather(table_v, [loc], mask=m)
        bad = m & (got != jv)
        n = plsc.all_reduce_population_count(bad)[0]

        def repair(_):
            def rbody(k, c):
                g = plsc.load_gather(table_v, [loc], mask=m)
                b = m & (g < jv)
                plsc.store_scatter(table_v, [loc], jv, mask=b)
                return c

            return lax.fori_loop(0, 15, rbody, 0)

        lax.cond(n > 0, repair, lambda _: 0, 0)
        return carry

    lax.fori_loop(0, B // 16, body, 0)
    pltpu.sync_copy(table_v, table_hbm.at[pl.ds(lo, RP)])


@functools.partial(
    pl.kernel,
    out_type=jax.ShapeDtypeStruct((NW, NCH, CH), jnp.float32),
    mesh=_mesh,
    scratch_types=[
        pltpu.VMEM((NCH, CH), jnp.int32),
        pltpu.VMEM((NCH, CH), jnp.int32),
        pltpu.VMEM((NCH, CH), jnp.float32),
        pltpu.SemaphoreType.DMA,
    ],
    compiler_params=pltpu.CompilerParams(needs_layout_passes=False),
)
def _emit(idx_hbm, val_hbm, table_hbm, out_hbm, idx_v, w_v, val_v, sem):
    wid = lax.axis_index("s") * NC + lax.axis_index("c")
    pltpu.sync_copy(idx_hbm.at[wid], idx_v)
    for j in range(NCH):
        pltpu.async_copy(table_hbm.at[idx_v.at[j]], w_v.at[j], sem).wait()
    for j in range(NCH):
        pltpu.async_copy(val_hbm.at[w_v.at[j]], val_v.at[j], sem).wait()
    pltpu.sync_copy(val_v, out_hbm.at[wid])


def kernel(node_idxs, values, state):
    idx_flat = node_idxs.astype(jnp.int32)
    idx_3d = idx_flat.reshape(NW, NCH, CH)
    val_flat = values.reshape(B)
    table = _winners(idx_flat)
    out = _emit(idx_3d, val_flat, table)
    return out.reshape(B, 1)


# speedup vs baseline: 3.2938x; 3.2938x over previous
import functools

import jax
import jax.numpy as jnp
from jax import lax
from jax.experimental import pallas as pl
from jax.experimental.pallas import tpu as pltpu
from jax.experimental.pallas import tpu_sc as plsc

B = 16384
N = 1000000
NC = 2
NS = 16
NW = NC * NS
RP = 31264
NT = NW * RP
BW = B // NW
CH = 128
NCH = BW // CH

_mesh = plsc.VectorSubcoreMesh(core_axis_name="c", subcore_axis_name="s")


@functools.partial(
    pl.kernel,
    out_type=jax.ShapeDtypeStruct((NT,), jnp.int32),
    mesh=_mesh,
    scratch_types=[
        pltpu.VMEM((B,), jnp.int32),
        pltpu.VMEM((RP,), jnp.int32),
    ],
    compiler_params=pltpu.CompilerParams(needs_layout_passes=False),
)
def _winners(idx_hbm, table_hbm, idx_v, table_v):
    wid = lax.axis_index("s") * NC + lax.axis_index("c")
    lo = wid * RP
    pltpu.sync_copy(idx_hbm, idx_v)
    iota = lax.iota(jnp.int32, 16)
    U = 8

    def body(vb, carry):
        base = vb * (16 * U)
        for u in range(U):
            iv = idx_v[pl.ds(base + u * 16, 16)]
            loc = iv - lo
            m = (loc >= 0) & (loc < RP)
            _, keep = plsc.scan_count(loc, m)
            plsc.store_scatter(table_v, [loc], base + u * 16 + iota, mask=keep)
        return carry

    lax.fori_loop(0, B // (16 * U), body, 0)
    pltpu.sync_copy(table_v, table_hbm.at[pl.ds(lo, RP)])


@functools.partial(
    pl.kernel,
    out_type=jax.ShapeDtypeStruct((NW, NCH, CH), jnp.float32),
    mesh=_mesh,
    scratch_types=[
        pltpu.VMEM((NCH, CH), jnp.int32),
        pltpu.VMEM((NCH, CH), jnp.int32),
        pltpu.VMEM((NCH, CH), jnp.float32),
        pltpu.SemaphoreType.DMA,
    ],
    compiler_params=pltpu.CompilerParams(needs_layout_passes=False),
)
def _emit(idx_hbm, val_hbm, table_hbm, out_hbm, idx_v, w_v, val_v, sem):
    wid = lax.axis_index("s") * NC + lax.axis_index("c")
    pltpu.sync_copy(idx_hbm.at[wid], idx_v)
    for j in range(NCH):
        pltpu.async_copy(table_hbm.at[idx_v.at[j]], w_v.at[j], sem).wait()
    for j in range(NCH):
        pltpu.async_copy(val_hbm.at[w_v.at[j]], val_v.at[j], sem).wait()
    pltpu.sync_copy(val_v, out_hbm.at[wid])


def kernel(node_idxs, values, state):
    idx_flat = node_idxs.astype(jnp.int32)
    idx_3d = idx_flat.reshape(NW, NCH, CH)
    val_flat = values.reshape(B)
    table = _winners(idx_flat)
    out = _emit(idx_3d, val_flat, table)
    return out.reshape(B, 1)


# speedup vs baseline: 3.5353x; 1.0733x over previous
import functools

import jax
import jax.numpy as jnp
from jax import lax
from jax.experimental import pallas as pl
from jax.experimental.pallas import tpu as pltpu
from jax.experimental.pallas import tpu_sc as plsc

B = 16384
N = 1000000
NC = 2
NS = 16
RP = 62592
NT = NS * RP
BW = B // (NC * NS)
CH = 128
NCH = BW // CH
U = 8

_mesh = plsc.VectorSubcoreMesh(core_axis_name="c", subcore_axis_name="s")


@functools.partial(
    pl.kernel,
    out_type=(jax.ShapeDtypeStruct((B,), jnp.float32),
              jax.ShapeDtypeStruct((NC * NT,), jnp.int32)),
    mesh=_mesh,
    scratch_types=[
        pltpu.VMEM((B,), jnp.int32),
        pltpu.VMEM((RP,), jnp.int32),
        pltpu.VMEM((BW,), jnp.int32),
        pltpu.VMEM((BW,), jnp.float32),
        pltpu.SemaphoreType.DMA,
    ],
    compiler_params=pltpu.CompilerParams(needs_layout_passes=False),
)
def _state_gather(idx_hbm, val_hbm, out_hbm, tabs_hbm, idx_v, table_v, w_v, val_v, sem):
    cid = lax.axis_index("c")
    sid = lax.axis_index("s")
    lo = pl.multiple_of(sid * RP, 128)
    pltpu.sync_copy(idx_hbm, idx_v)
    iota = lax.iota(jnp.int32, 16)
    rp_u = jnp.uint32(RP)

    def body(vb, carry):
        base = vb * (16 * U)
        for u in range(U):
            iv = idx_v[pl.ds(base + u * 16, 16)]
            _, keep = plsc.scan_count(iv)
            loc = iv - lo
            m = loc.astype(jnp.uint32) < rp_u
            plsc.store_scatter(table_v, [loc], base + (u * 16 + iota), mask=m & keep)
        return carry

    lax.fori_loop(0, B // (16 * U), body, 0)
    stab = tabs_hbm.at[pl.ds(pl.multiple_of(cid * NT, 128), NT)]
    pltpu.sync_copy(table_v, stab.at[pl.ds(lo, RP)])
    plsc.subcore_barrier()

    base = pl.multiple_of((cid * NS + sid) * BW, 128)
    for j in range(NCH):
        pltpu.async_copy(
            stab.at[idx_v.at[pl.ds(base + j * CH, CH)]],
            w_v.at[pl.ds(j * CH, CH)], sem).wait()
    for j in range(NCH):
        pltpu.async_copy(
            val_hbm.at[w_v.at[pl.ds(j * CH, CH)]],
            val_v.at[pl.ds(j * CH, CH)], sem).wait()
    pltpu.sync_copy(val_v, out_hbm.at[pl.ds(base, BW)])


def kernel(node_idxs, values, state):
    idx_flat = node_idxs.astype(jnp.int32)
    val_flat = values.reshape(B)
    out, _ = _state_gather(idx_flat, val_flat)
    return out.reshape(B, 1)


# speedup vs baseline: 4.1275x; 1.1675x over previous
import functools

import jax
import jax.numpy as jnp
from jax import lax
from jax.experimental import pallas as pl
from jax.experimental.pallas import tpu as pltpu
from jax.experimental.pallas import tpu_sc as plsc

B = 16384
N = 1000000
NC = 2
NS = 16
RP = 62592
NT = NS * RP
BW = B // (NC * NS)
CH = 128
NCH = BW // CH
U = 8

_mesh = plsc.VectorSubcoreMesh(core_axis_name="c", subcore_axis_name="s")


@functools.partial(
    pl.kernel,
    out_type=(jax.ShapeDtypeStruct((B,), jnp.float32),
              jax.ShapeDtypeStruct((NC * NT,), jnp.int32)),
    mesh=_mesh,
    scratch_types=[
        pltpu.VMEM((B,), jnp.int32),
        pltpu.VMEM((RP,), jnp.int32),
        pltpu.VMEM((BW,), jnp.int32),
        pltpu.VMEM((BW,), jnp.int32),
        pltpu.VMEM((BW,), jnp.float32),
        pltpu.SemaphoreType.DMA,
    ],
    compiler_params=pltpu.CompilerParams(needs_layout_passes=False),
)
def _state_gather(idx_hbm, val_hbm, out_hbm, tabs_hbm, idx_v, table_v, w_v,
                  j_v, val_v, sem):
    cid = lax.axis_index("c")
    sid = lax.axis_index("s")
    lo = pl.multiple_of(sid * RP, 128)
    pltpu.sync_copy(idx_hbm, idx_v)
    rp_u = jnp.uint32(RP)

    def body(vb, carry):
        base = vb * (16 * U)
        for u in range(U):
            iv = idx_v[pl.ds(base + u * 16, 16)]
            loc = iv - lo
            m = loc.astype(jnp.uint32) < rp_u
            plsc.store_scatter(table_v, [loc],
                               jnp.full((16,), vb * U + u, jnp.int32), mask=m)
        return carry

    lax.fori_loop(0, B // (16 * U), body, 0)
    stab = tabs_hbm.at[pl.ds(pl.multiple_of(cid * NT, 128), NT)]
    pltpu.sync_copy(table_v, stab.at[pl.ds(lo, RP)])
    plsc.subcore_barrier()

    base = pl.multiple_of((cid * NS + sid) * BW, 128)
    for j in range(NCH):
        pltpu.async_copy(
            stab.at[idx_v.at[pl.ds(base + j * CH, CH)]],
            w_v.at[pl.ds(j * CH, CH)], sem).wait()

    def resolve(k, carry):
        myidx = idx_v[pl.ds(base + k * 16, 16)]
        wv16 = w_v[pl.ds(k * 16, 16)] * 16
        jbest = wv16
        for l in range(16):
            cand = plsc.load_gather(idx_v, [wv16 + l])
            jbest = jnp.where(cand == myidx, wv16 + l, jbest)
        j_v[pl.ds(k * 16, 16)] = jbest
        return carry

    lax.fori_loop(0, BW // 16, resolve, 0)

    for j in range(NCH):
        pltpu.async_copy(
            val_hbm.at[j_v.at[pl.ds(j * CH, CH)]],
            val_v.at[pl.ds(j * CH, CH)], sem).wait()
    pltpu.sync_copy(val_v, out_hbm.at[pl.ds(base, BW)])


def kernel(node_idxs, values, state):
    idx_flat = node_idxs.astype(jnp.int32)
    val_flat = values.reshape(B)
    out, _ = _state_gather(idx_flat, val_flat)
    return out.reshape(B, 1)


# speedup vs baseline: 4.5905x; 1.1122x over previous
import functools

import jax
import jax.numpy as jnp
from jax import lax
from jax.experimental import pallas as pl
from jax.experimental.pallas import tpu as pltpu
from jax.experimental.pallas import tpu_sc as plsc

B = 16384
N = 1000000
NC = 2
NS = 16
RP = 62592
NT = NS * RP
BW = B // (NC * NS)
CH = 128
NCH = BW // CH
U = 8

_mesh = plsc.VectorSubcoreMesh(core_axis_name="c", subcore_axis_name="s")


@functools.partial(
    pl.kernel,
    out_type=(jax.ShapeDtypeStruct((B,), jnp.float32),
              jax.ShapeDtypeStruct((NC * NT,), jnp.int32)),
    mesh=_mesh,
    scratch_types=[
        pltpu.VMEM((B,), jnp.int32),
        pltpu.VMEM((RP,), jnp.int32),
        pltpu.VMEM((BW,), jnp.int32),
        pltpu.VMEM((BW,), jnp.int32),
        pltpu.VMEM((BW,), jnp.float32),
        pltpu.SemaphoreType.DMA,
    ],
    compiler_params=pltpu.CompilerParams(needs_layout_passes=False),
)
def _state_gather(idx_hbm, val_hbm, out_hbm, tabs_hbm, idx_v, table_v, w_v,
                  j_v, val_v, sem):
    cid = lax.axis_index("c")
    sid = lax.axis_index("s")
    lo = pl.multiple_of(sid * RP, 128)
    with jax.named_scope("idx_in"):
        pltpu.sync_copy(idx_hbm, idx_v)
    rp_u = jnp.uint32(RP)

    def body(vb, carry):
        base = vb * (16 * U)
        for u in range(U):
            iv = idx_v[pl.ds(base + u * 16, 16)]
            loc = iv - lo
            m = loc.astype(jnp.uint32) < rp_u
            plsc.store_scatter(table_v, [loc],
                               jnp.full((16,), vb * U + u, jnp.int32), mask=m)
        return carry

    with jax.named_scope("scan"):
        lax.fori_loop(0, B // (16 * U), body, 0)
    stab = tabs_hbm.at[pl.ds(pl.multiple_of(cid * NT, 128), NT)]
    with jax.named_scope("dump"):
        pltpu.sync_copy(table_v, stab.at[pl.ds(lo, RP)])
        plsc.subcore_barrier()

    base = pl.multiple_of((cid * NS + sid) * BW, 128)
    with jax.named_scope("wgather"):
        copies = [
            pltpu.async_copy(
                stab.at[idx_v.at[pl.ds(base + j * CH, CH)]],
                w_v.at[pl.ds(j * CH, CH)], sem)
            for j in range(NCH)
        ]
        for c in copies:
            c.wait()

    def resolve(k, carry):
        myidx = idx_v[pl.ds(base + k * 16, 16)]
        wv16 = w_v[pl.ds(k * 16, 16)] * 16
        jbest = wv16
        for l in range(16):
            cand = plsc.load_gather(idx_v, [wv16 + l])
            jbest = jnp.where(cand == myidx, wv16 + l, jbest)
        j_v[pl.ds(k * 16, 16)] = jbest
        return carry

    with jax.named_scope("resolve"):
        lax.fori_loop(0, BW // 16, resolve, 0)

    with jax.named_scope("vgather"):
        copies = [
            pltpu.async_copy(
                val_hbm.at[j_v.at[pl.ds(j * CH, CH)]],
                val_v.at[pl.ds(j * CH, CH)], sem)
            for j in range(NCH)
        ]
        for c in copies:
            c.wait()
    with jax.named_scope("out"):
        pltpu.sync_copy(val_v, out_hbm.at[pl.ds(base, BW)])


def kernel(node_idxs, values, state):
    idx_flat = node_idxs.astype(jnp.int32)
    val_flat = values.reshape(B)
    out, _ = _state_gather(idx_flat, val_flat)
    return out.reshape(B, 1)


# speedup vs baseline: 5.4471x; 1.1866x over previous
import functools

import jax
import jax.numpy as jnp
from jax import lax
from jax.experimental import pallas as pl
from jax.experimental.pallas import tpu as pltpu
from jax.experimental.pallas import tpu_sc as plsc

B = 16384
N = 1000000
NC = 2
NS = 16
RP = 62592
NT = NS * RP
BW = B // (NC * NS)
CH = 128
NCH = BW // CH
U = 8

_mesh = plsc.VectorSubcoreMesh(core_axis_name="c", subcore_axis_name="s")


@functools.partial(
    pl.kernel,
    out_type=(jax.ShapeDtypeStruct((B,), jnp.float32),
              jax.ShapeDtypeStruct((NC * NT,), jnp.int32)),
    mesh=_mesh,
    scratch_types=[
        pltpu.VMEM((B,), jnp.int32),
        pltpu.VMEM((RP,), jnp.int32),
        pltpu.VMEM((BW,), jnp.int32),
        pltpu.VMEM((BW,), jnp.int32),
        pltpu.VMEM((BW,), jnp.float32),
        pltpu.SemaphoreType.DMA,
    ],
    compiler_params=pltpu.CompilerParams(needs_layout_passes=False),
)
def _state_gather(idx_hbm, val_hbm, out_hbm, tabs_hbm, idx_v, table_v, w_v,
                  j_v, val_v, sem):
    cid = lax.axis_index("c")
    sid = lax.axis_index("s")
    lo = pl.multiple_of(sid * RP, 128)
    with jax.named_scope("idx_in"):
        pltpu.sync_copy(idx_hbm, idx_v)
    rp_u = jnp.uint32(RP)

    def body(vb, carry):
        base = vb * (16 * U)
        ivs = [idx_v[pl.ds(base + u * 16, 16)] for u in range(U)]
        locs = [iv - lo for iv in ivs]
        ms = [loc.astype(jnp.uint32) < rp_u for loc in locs]
        for u in range(U):
            plsc.store_scatter(table_v, [locs[u]],
                               jnp.full((16,), vb * U + u, jnp.int32), mask=ms[u])
        return carry

    with jax.named_scope("scan"):
        lax.fori_loop(0, B // (16 * U), body, 0)
    stab = tabs_hbm.at[pl.ds(pl.multiple_of(cid * NT, 128), NT)]
    with jax.named_scope("dump"):
        pltpu.sync_copy(table_v, stab.at[pl.ds(lo, RP)])
        plsc.subcore_barrier()

    base = pl.multiple_of((cid * NS + sid) * BW, 128)
    with jax.named_scope("wgather"):
        copies = [
            pltpu.async_copy(
                stab.at[idx_v.at[pl.ds(base + j * CH, CH)]],
                w_v.at[pl.ds(j * CH, CH)], sem)
            for j in range(NCH)
        ]
        for c in copies:
            c.wait()

    def resolve(k, carry):
        myidx = idx_v[pl.ds(base + k * 16, 16)]
        wv16 = w_v[pl.ds(k * 16, 16)] * 16
        jbest = wv16
        for l in range(16):
            cand = plsc.load_gather(idx_v, [wv16 + l])
            jbest = jnp.where(cand == myidx, wv16 + l, jbest)
        j_v[pl.ds(k * 16, 16)] = jbest
        return carry

    with jax.named_scope("resolve"):
        lax.fori_loop(0, BW // 16, resolve, 0)

    with jax.named_scope("vgather"):
        copies = [
            pltpu.async_copy(
                val_hbm.at[j_v.at[pl.ds(j * CH, CH)]],
                val_v.at[pl.ds(j * CH, CH)], sem)
            for j in range(NCH)
        ]
        for c in copies:
            c.wait()
    with jax.named_scope("out"):
        pltpu.sync_copy(val_v, out_hbm.at[pl.ds(base, BW)])


def kernel(node_idxs, values, state):
    idx_flat = node_idxs.astype(jnp.int32)
    val_flat = values.reshape(B)
    out, _ = _state_gather(idx_flat, val_flat)
    return out.reshape(B, 1)
